# baseline (speedup 1.0000x reference)
import functools

import jax
import jax.numpy as jnp
from jax import lax
from jax.experimental import pallas as pl
from jax.experimental.pallas import tpu as pltpu
from jax.experimental.pallas import tpu_sc as plsc

_N = 10000
_E = 320000
_D = 128
_OUT = 64

_NP = 10240
_ROWS_PER_SUB = _NP // 16
_CH = 128
_NTILES = 32
_NCH = 81
_EPAD = _NTILES * _NCH * _CH


def _sc_aggregate(x, src3, dst3, zrow, zdeg, ones):
    mesh = plsc.VectorSubcoreMesh(core_axis_name="c", subcore_axis_name="s")

    @functools.partial(
        pl.kernel,
        mesh=mesh,
        out_type=[
            jax.ShapeDtypeStruct((2, _NP, _D), jnp.float32),
            jax.ShapeDtypeStruct((2, _NP), jnp.float32),
        ],
        scratch_types=[
            pltpu.VMEM((_NCH, _CH), jnp.int32),
            pltpu.VMEM((_NCH, _CH), jnp.int32),
            pltpu.VMEM((_CH, _D), jnp.float32),
            pltpu.VMEM((_CH,), jnp.float32),
            pltpu.VMEM_SHARED((_NP, _D), jnp.float32),
            pltpu.VMEM_SHARED((_NP,), jnp.float32),
            pltpu.SemaphoreType.DMA,
        ],
    )
    def k(x_hbm, src_hbm, dst_hbm, zrow_hbm, zdeg_hbm, ones_hbm,
          agg_out, deg_out, sidx, didx, rows, onesv, aggsh, degsh, sem):
        c = lax.axis_index("c")
        s = lax.axis_index("s")
        w = c * 16 + s
        r0 = s * _ROWS_PER_SUB
        pltpu.sync_copy(zrow_hbm, aggsh.at[pl.ds(r0, _ROWS_PER_SUB)])
        pltpu.sync_copy(zdeg_hbm, degsh.at[pl.ds(r0, _ROWS_PER_SUB)])
        pltpu.sync_copy(src_hbm.at[w], sidx)
        pltpu.sync_copy(dst_hbm.at[w], didx)
        pltpu.sync_copy(ones_hbm, onesv)
        plsc.subcore_barrier()

        def body(j, carry):
            pltpu.async_copy(x_hbm.at[sidx.at[j]], rows, sem).wait()
            pltpu.sync_copy(rows, aggsh.at[didx.at[j]], add=True)
            pltpu.sync_copy(onesv, degsh.at[didx.at[j]], add=True)
            return carry

        lax.fori_loop(0, _NCH, body, 0)
        plsc.subcore_barrier()
        pltpu.sync_copy(aggsh.at[pl.ds(r0, _ROWS_PER_SUB)],
                        agg_out.at[c, pl.ds(r0, _ROWS_PER_SUB)])
        pltpu.sync_copy(degsh.at[pl.ds(r0, _ROWS_PER_SUB)],
                        deg_out.at[c, pl.ds(r0, _ROWS_PER_SUB)])

    return k(x, src3, dst3, zrow, zdeg, ones)


def _tc_body(x_ref, aggp_ref, degp_ref, eps_ref,
             w1_ref, b1_ref, g1_ref, be1_ref,
             w2_ref, b2_ref, g2_ref, be2_ref,
             wp0_ref, bp0_ref, wc_ref, bc_ref, out_ref):
    xv = x_ref[...]
    agg = aggp_ref[0, :_N, :] + aggp_ref[1, :_N, :]
    deg = degp_ref[0, :_N, :] + degp_ref[1, :_N, :]
    agg = agg / jnp.maximum(deg, 1.0)
    rst = (1.0 + eps_ref[0, 0]) * xv + agg
    t = jnp.dot(rst, w1_ref[...], preferred_element_type=jnp.float32) + b1_ref[...]
    mu = jnp.mean(t, axis=0, keepdims=True)
    var = jnp.mean((t - mu) ** 2, axis=0, keepdims=True)
    h = jnp.maximum((t - mu) / jnp.sqrt(var + 1e-5) * g1_ref[...] + be1_ref[...], 0.0)
    t2 = jnp.dot(h, w2_ref[...], preferred_element_type=jnp.float32) + b2_ref[...]
    mu2 = jnp.mean(t2, axis=0, keepdims=True)
    var2 = jnp.mean((t2 - mu2) ** 2, axis=0, keepdims=True)
    h2 = jnp.maximum((t2 - mu2) / jnp.sqrt(var2 + 1e-5) * g2_ref[...] + be2_ref[...], 0.0)
    s2 = jnp.sum(h2, axis=0, keepdims=True)
    sx = jnp.sum(xv, axis=0, keepdims=True)
    out0 = jnp.dot(sx, wp0_ref[...], preferred_element_type=jnp.float32) + bp0_ref[...]
    out1 = jnp.dot(s2, wc_ref[...], preferred_element_type=jnp.float32) + bc_ref[...]
    out_ref[...] = out0 + out1


def kernel(x, edge_index, eps0, W1, b1, g1, be1, W2, b2, g2, be2, Wp0, bp0, Wc, bc):
    pad = _EPAD - _E
    src3 = jnp.pad(edge_index[0], (0, pad)).reshape(_NTILES, _NCH, _CH)
    dummy_dst = _N + (jnp.arange(pad, dtype=jnp.int32) % (_NP - _N))
    dst3 = jnp.concatenate([edge_index[1], dummy_dst]).reshape(_NTILES, _NCH, _CH)
    zrow = jnp.zeros((_ROWS_PER_SUB, _D), jnp.float32)
    zdeg = jnp.zeros((_ROWS_PER_SUB,), jnp.float32)
    ones = jnp.ones((_CH,), jnp.float32)
    aggp, degp = _sc_aggregate(x, src3, dst3, zrow, zdeg, ones)

    out = pl.pallas_call(
        _tc_body,
        out_shape=jax.ShapeDtypeStruct((1, _OUT), jnp.float32),
    )(x, aggp, degp.reshape(2, _NP, 1), eps0.reshape(1, 1),
      W1, b1.reshape(1, _D), g1.reshape(1, _D), be1.reshape(1, _D),
      W2, b2.reshape(1, _D), g2.reshape(1, _D), be2.reshape(1, _D),
      Wp0, bp0.reshape(1, _OUT), Wc, bc.reshape(1, _OUT))
    return out


# speedup vs baseline: 3.1731x; 3.1731x over previous
import functools

import jax
import jax.numpy as jnp
from jax import lax
from jax.experimental import pallas as pl
from jax.experimental.pallas import tpu as pltpu
from jax.experimental.pallas import tpu_sc as plsc

_N = 10000
_E = 320000
_D = 128
_OUT = 64

_NP = 10240
_ROWS_PER_SUB = _NP // 16
_CH = 128
_NTILES = 32
_NCH = 81
_EPAD = _NTILES * _NCH * _CH


def _sc_aggregate(x, src3, dst3, zrow, zdeg, ones):
    mesh = plsc.VectorSubcoreMesh(core_axis_name="c", subcore_axis_name="s")

    @functools.partial(
        pl.kernel,
        mesh=mesh,
        out_type=[
            jax.ShapeDtypeStruct((2, _NP, _D), jnp.float32),
            jax.ShapeDtypeStruct((2, _NP), jnp.float32),
        ],
        scratch_types=[
            pltpu.VMEM((_NCH, _CH), jnp.int32),
            pltpu.VMEM((_NCH, _CH), jnp.int32),
            pltpu.VMEM((_CH, _D), jnp.float32),
            pltpu.VMEM((_CH,), jnp.float32),
            pltpu.VMEM_SHARED((_NP, _D), jnp.float32),
            pltpu.VMEM_SHARED((_NP,), jnp.float32),
            pltpu.SemaphoreType.DMA,
        ],
    )
    def k(x_hbm, src_hbm, dst_hbm, zrow_hbm, zdeg_hbm, ones_hbm,
          agg_out, deg_out, sidx, didx, rows, onesv, aggsh, degsh, sem):
        c = lax.axis_index("c")
        s = lax.axis_index("s")
        w = c * 16 + s
        r0 = s * _ROWS_PER_SUB
        pltpu.sync_copy(zrow_hbm, aggsh.at[pl.ds(r0, _ROWS_PER_SUB)])
        pltpu.sync_copy(zdeg_hbm, degsh.at[pl.ds(r0, _ROWS_PER_SUB)])
        pltpu.sync_copy(src_hbm.at[w], sidx)
        pltpu.sync_copy(dst_hbm.at[w], didx)
        pltpu.sync_copy(ones_hbm, onesv)
        plsc.subcore_barrier()

        def body(j, carry):
            pltpu.async_copy(x_hbm.at[sidx.at[j]], rows, sem).wait()
            pltpu.sync_copy(rows, aggsh.at[didx.at[j]], add=True)
            pltpu.sync_copy(onesv, degsh.at[didx.at[j]], add=True)
            return carry

        lax.fori_loop(0, _NCH, body, 0)
        plsc.subcore_barrier()
        pltpu.sync_copy(aggsh.at[pl.ds(r0, _ROWS_PER_SUB)],
                        agg_out.at[c, pl.ds(r0, _ROWS_PER_SUB)])
        pltpu.sync_copy(degsh.at[pl.ds(r0, _ROWS_PER_SUB)],
                        deg_out.at[c, pl.ds(r0, _ROWS_PER_SUB)])

    return k(x, src3, dst3, zrow, zdeg, ones)


def _tc_body(x_ref, aggp_ref, degp_ref, eps_ref,
             w1_ref, b1_ref, g1_ref, be1_ref,
             w2_ref, b2_ref, g2_ref, be2_ref,
             wp0_ref, bp0_ref, wc_ref, bc_ref, out_ref):
    xv = x_ref[...]
    agg = aggp_ref[0, :_N, :] + aggp_ref[1, :_N, :]
    deg = degp_ref[0, :_N, :] + degp_ref[1, :_N, :]
    agg = agg / jnp.maximum(deg, 1.0)
    rst = (1.0 + eps_ref[0, 0]) * xv + agg
    t = jnp.dot(rst, w1_ref[...], preferred_element_type=jnp.float32) + b1_ref[...]
    mu = jnp.mean(t, axis=0, keepdims=True)
    var = jnp.mean((t - mu) ** 2, axis=0, keepdims=True)
    h = jnp.maximum((t - mu) / jnp.sqrt(var + 1e-5) * g1_ref[...] + be1_ref[...], 0.0)
    t2 = jnp.dot(h, w2_ref[...], preferred_element_type=jnp.float32) + b2_ref[...]
    mu2 = jnp.mean(t2, axis=0, keepdims=True)
    var2 = jnp.mean((t2 - mu2) ** 2, axis=0, keepdims=True)
    h2 = jnp.maximum((t2 - mu2) / jnp.sqrt(var2 + 1e-5) * g2_ref[...] + be2_ref[...], 0.0)
    s2 = jnp.sum(h2, axis=0, keepdims=True)
    sx = jnp.sum(xv, axis=0, keepdims=True)
    out0 = jnp.dot(sx, wp0_ref[...], preferred_element_type=jnp.float32) + bp0_ref[...]
    out1 = jnp.dot(s2, wc_ref[...], preferred_element_type=jnp.float32) + bc_ref[...]
    out_ref[...] = out0 + out1


def kernel(x, edge_index, eps0, W1, b1, g1, be1, W2, b2, g2, be2, Wp0, bp0, Wc, bc):
    pad = _EPAD - _E
    dummy_src = jnp.arange(pad, dtype=jnp.int32) % _N
    src3 = jnp.concatenate([edge_index[0], dummy_src]).reshape(_NTILES, _NCH, _CH)
    dummy_dst = _N + (jnp.arange(pad, dtype=jnp.int32) % (_NP - _N))
    dst3 = jnp.concatenate([edge_index[1], dummy_dst]).reshape(_NTILES, _NCH, _CH)
    zrow = jnp.zeros((_ROWS_PER_SUB, _D), jnp.float32)
    zdeg = jnp.zeros((_ROWS_PER_SUB,), jnp.float32)
    ones = jnp.ones((_CH,), jnp.float32)
    aggp, degp = _sc_aggregate(x, src3, dst3, zrow, zdeg, ones)

    out = pl.pallas_call(
        _tc_body,
        out_shape=jax.ShapeDtypeStruct((1, _OUT), jnp.float32),
    )(x, aggp, degp.reshape(2, _NP, 1), eps0.reshape(1, 1),
      W1, b1.reshape(1, _D), g1.reshape(1, _D), be1.reshape(1, _D),
      W2, b2.reshape(1, _D), g2.reshape(1, _D), be2.reshape(1, _D),
      Wp0, bp0.reshape(1, _OUT), Wc, bc.reshape(1, _OUT))
    return out


# speedup vs baseline: 3.2194x; 1.0146x over previous
import functools

import jax
import jax.numpy as jnp
from jax import lax
from jax.experimental import pallas as pl
from jax.experimental.pallas import tpu as pltpu
from jax.experimental.pallas import tpu_sc as plsc

_N = 10000
_E = 320000
_D = 128
_OUT = 64

_NP = 10240
_ROWS_PER_SUB = _NP // 16
_CH = 128
_NTILES = 32
_NCH = 79
_EPAD = _NTILES * _NCH * _CH


def _sc_aggregate(x, src3, dst3, zrow, zdeg, ones):
    mesh = plsc.VectorSubcoreMesh(core_axis_name="c", subcore_axis_name="s")

    @functools.partial(
        pl.kernel,
        mesh=mesh,
        out_type=[
            jax.ShapeDtypeStruct((2, _NP, _D), jnp.float32),
            jax.ShapeDtypeStruct((2, _NP), jnp.float32),
        ],
        scratch_types=[
            pltpu.VMEM((_NCH, _CH), jnp.int32),
            pltpu.VMEM((_NCH, _CH), jnp.int32),
            pltpu.VMEM((_CH, _D), jnp.float32),
            pltpu.VMEM((_CH,), jnp.float32),
            pltpu.VMEM_SHARED((_NP, _D), jnp.float32),
            pltpu.VMEM_SHARED((_NP,), jnp.float32),
            pltpu.SemaphoreType.DMA,
        ],
    )
    def k(x_hbm, src_hbm, dst_hbm, zrow_hbm, zdeg_hbm, ones_hbm,
          agg_out, deg_out, sidx, didx, rows, onesv, aggsh, degsh, sem):
        c = lax.axis_index("c")
        s = lax.axis_index("s")
        w = c * 16 + s
        r0 = s * _ROWS_PER_SUB
        pltpu.sync_copy(zrow_hbm, aggsh.at[pl.ds(r0, _ROWS_PER_SUB)])
        pltpu.sync_copy(zdeg_hbm, degsh.at[pl.ds(r0, _ROWS_PER_SUB)])
        pltpu.sync_copy(src_hbm.at[w], sidx)
        pltpu.sync_copy(dst_hbm.at[w], didx)
        pltpu.sync_copy(ones_hbm, onesv)
        plsc.subcore_barrier()

        def body(j, carry):
            pltpu.async_copy(x_hbm.at[sidx.at[j]], rows, sem).wait()
            pltpu.sync_copy(rows, aggsh.at[didx.at[j]], add=True)
            pltpu.sync_copy(onesv, degsh.at[didx.at[j]], add=True)
            return carry

        lax.fori_loop(0, _NCH, body, 0)
        plsc.subcore_barrier()
        pltpu.sync_copy(aggsh.at[pl.ds(r0, _ROWS_PER_SUB)],
                        agg_out.at[c, pl.ds(r0, _ROWS_PER_SUB)])
        pltpu.sync_copy(degsh.at[pl.ds(r0, _ROWS_PER_SUB)],
                        deg_out.at[c, pl.ds(r0, _ROWS_PER_SUB)])

    return k(x, src3, dst3, zrow, zdeg, ones)


def _tc_body(x_ref, aggp_ref, degp_ref, eps_ref,
             w1_ref, b1_ref, g1_ref, be1_ref,
             w2_ref, b2_ref, g2_ref, be2_ref,
             wp0_ref, bp0_ref, wc_ref, bc_ref, out_ref):
    xv = x_ref[...]
    agg = aggp_ref[0, :_N, :] + aggp_ref[1, :_N, :]
    deg = degp_ref[0, :_N, :] + degp_ref[1, :_N, :]
    agg = agg / jnp.maximum(deg, 1.0)
    rst = (1.0 + eps_ref[0, 0]) * xv + agg
    t = jnp.dot(rst, w1_ref[...], preferred_element_type=jnp.float32) + b1_ref[...]
    mu = jnp.mean(t, axis=0, keepdims=True)
    var = jnp.mean((t - mu) ** 2, axis=0, keepdims=True)
    h = jnp.maximum((t - mu) / jnp.sqrt(var + 1e-5) * g1_ref[...] + be1_ref[...], 0.0)
    t2 = jnp.dot(h, w2_ref[...], preferred_element_type=jnp.float32) + b2_ref[...]
    mu2 = jnp.mean(t2, axis=0, keepdims=True)
    var2 = jnp.mean((t2 - mu2) ** 2, axis=0, keepdims=True)
    h2 = jnp.maximum((t2 - mu2) / jnp.sqrt(var2 + 1e-5) * g2_ref[...] + be2_ref[...], 0.0)
    s2 = jnp.sum(h2, axis=0, keepdims=True)
    sx = jnp.sum(xv, axis=0, keepdims=True)
    out0 = jnp.dot(sx, wp0_ref[...], preferred_element_type=jnp.float32) + bp0_ref[...]
    out1 = jnp.dot(s2, wc_ref[...], preferred_element_type=jnp.float32) + bc_ref[...]
    out_ref[...] = out0 + out1


def kernel(x, edge_index, eps0, W1, b1, g1, be1, W2, b2, g2, be2, Wp0, bp0, Wc, bc):
    pad = _EPAD - _E
    dummy_src = jnp.arange(pad, dtype=jnp.int32) % _N
    src3 = jnp.concatenate([edge_index[0], dummy_src]).reshape(_NTILES, _NCH, _CH)
    dummy_dst = _N + (jnp.arange(pad, dtype=jnp.int32) % (_NP - _N))
    dst3 = jnp.concatenate([edge_index[1], dummy_dst]).reshape(_NTILES, _NCH, _CH)
    zrow = jnp.zeros((_ROWS_PER_SUB, _D), jnp.float32)
    zdeg = jnp.zeros((_ROWS_PER_SUB,), jnp.float32)
    ones = jnp.ones((_CH,), jnp.float32)
    aggp, degp = _sc_aggregate(x, src3, dst3, zrow, zdeg, ones)

    out = pl.pallas_call(
        _tc_body,
        out_shape=jax.ShapeDtypeStruct((1, _OUT), jnp.float32),
    )(x, aggp, degp.reshape(2, _NP, 1), eps0.reshape(1, 1),
      W1, b1.reshape(1, _D), g1.reshape(1, _D), be1.reshape(1, _D),
      W2, b2.reshape(1, _D), g2.reshape(1, _D), be2.reshape(1, _D),
      Wp0, bp0.reshape(1, _OUT), Wc, bc.reshape(1, _OUT))
    return out
